# baseline (speedup 1.0000x reference)
import jax
import jax.numpy as jnp
from jax.experimental import pallas as pl


def _pos_kernel(row_ref, col_ref, out_ref, *, h, w, d):
    hw = h * w
    p = jax.lax.broadcasted_iota(jnp.int32, (max(h, w), hw), 1)
    q = jax.lax.broadcasted_iota(jnp.int32, (max(h, w), hw), 0)
    sel_col = (p % w == q).astype(jnp.float32)[:w, :]
    sel_row = (p // w == q).astype(jnp.float32)[:h, :]
    col = col_ref[:w, :]
    row = row_ref[:h, :]
    top = jax.lax.dot_general(col, sel_col, (((0,), (0,)), ((), ())),
                              preferred_element_type=jnp.float32)
    bottom = jax.lax.dot_general(row, sel_row, (((0,), (0,)), ((), ())),
                                 preferred_element_type=jnp.float32)
    out_ref[0, :d, :] = top
    out_ref[0, d:, :] = bottom


def kernel(x, row_embed, col_embed):
    b = x.shape[0]
    h, w = x.shape[2], x.shape[3]
    n, d = row_embed.shape
    import functools
    body = functools.partial(_pos_kernel, h=h, w=w, d=d)
    out = pl.pallas_call(
        body,
        grid=(b,),
        in_specs=[
            pl.BlockSpec((n, d), lambda i: (0, 0)),
            pl.BlockSpec((n, d), lambda i: (0, 0)),
        ],
        out_specs=pl.BlockSpec((1, 2 * d, h * w), lambda i: (i, 0, 0)),
        out_shape=jax.ShapeDtypeStruct((b, 2 * d, h * w), jnp.float32),
    )(row_embed, col_embed)
    return out.reshape(b, 2 * d, h, w)


# speedup vs baseline: 2.0231x; 2.0231x over previous
import functools

import jax
import jax.numpy as jnp
from jax.experimental import pallas as pl
from jax.experimental.pallas import tpu as pltpu


def _pos_kernel(row_ref, col_ref, out_ref, scratch, sem, *, b, h, w, d):
    hw = h * w
    p = jax.lax.broadcasted_iota(jnp.int32, (max(h, w), hw), 1)
    q = jax.lax.broadcasted_iota(jnp.int32, (max(h, w), hw), 0)
    sel_col = (p % w == q).astype(jnp.float32)[:w, :]
    sel_row = (p // w == q).astype(jnp.float32)[:h, :]
    scratch[:d, :] = jax.lax.dot_general(
        col_ref[:w, :], sel_col, (((0,), (0,)), ((), ())),
        preferred_element_type=jnp.float32,
        precision=jax.lax.Precision.HIGHEST)
    scratch[d:, :] = jax.lax.dot_general(
        row_ref[:h, :], sel_row, (((0,), (0,)), ((), ())),
        preferred_element_type=jnp.float32,
        precision=jax.lax.Precision.HIGHEST)
    for i in range(b):
        pltpu.make_async_copy(scratch, out_ref.at[i], sem).start()
    for i in range(b):
        pltpu.make_async_copy(scratch, out_ref.at[i], sem).wait()


def kernel(x, row_embed, col_embed):
    b = x.shape[0]
    h, w = x.shape[2], x.shape[3]
    n, d = row_embed.shape
    body = functools.partial(_pos_kernel, b=b, h=h, w=w, d=d)
    out = pl.pallas_call(
        body,
        in_specs=[
            pl.BlockSpec((n, d), lambda: (0, 0)),
            pl.BlockSpec((n, d), lambda: (0, 0)),
        ],
        out_specs=pl.BlockSpec(memory_space=pltpu.MemorySpace.HBM),
        out_shape=jax.ShapeDtypeStruct((b, 2 * d, h * w), jnp.float32),
        scratch_shapes=[
            pltpu.VMEM((2 * d, h * w), jnp.float32),
            pltpu.SemaphoreType.DMA,
        ],
    )(row_embed, col_embed)
    return out.reshape(b, 2 * d, h, w)
